# baseline (speedup 1.0000x reference)
import dataclasses
import functools

import jax
import jax.numpy as jnp
from jax import lax
from jax.experimental import pallas as pl
from jax.experimental.pallas import tpu as pltpu
from jax.experimental.pallas import tpu_sc as plsc

N = 10000
E = 320000
D = 128
NP = 10240
NW = 32
EPW = 10240
CH = 64
RC = EPW // CH
ROWS_PER_TILE = NP // 16

_mesh = plsc.VectorSubcoreMesh(core_axis_name="c", subcore_axis_name="s")

_sc_params = pltpu.CompilerParams()
if "needs_layout_passes" in pltpu.CompilerParams.__dataclass_fields__:
    _sc_params = dataclasses.replace(_sc_params, needs_layout_passes=False)


@functools.partial(
    pl.kernel,
    out_type=[
        jax.ShapeDtypeStruct((2, NP), jnp.float32),
        jax.ShapeDtypeStruct((NW, RC, CH), jnp.float32),
    ],
    mesh=_mesh,
    scratch_types=[
        pltpu.VMEM((RC, CH), jnp.int32),
        pltpu.VMEM((RC, CH), jnp.int32),
        pltpu.VMEM((RC, CH), jnp.float32),
        pltpu.VMEM((RC, CH), jnp.float32),
        pltpu.VMEM((RC, CH), jnp.float32),
        pltpu.VMEM((ROWS_PER_TILE,), jnp.float32),
        pltpu.VMEM_SHARED((NP,), jnp.float32),
    ],
)
def _degree_kernel(ii_hbm, jj_hbm, attr_hbm, degp_hbm, attrm_hbm,
                   ii_v, jj_v, attr_v, ones_v, attrm_v, zbuf, deg_acc):
    cid = lax.axis_index("c")
    sid = lax.axis_index("s")
    wid = cid * 16 + sid

    @pl.loop(0, ROWS_PER_TILE, step=16)
    def _(k):
        zbuf[pl.ds(k, 16)] = jnp.zeros((16,), jnp.float32)

    pltpu.sync_copy(zbuf, deg_acc.at[pl.ds(sid * ROWS_PER_TILE, ROWS_PER_TILE)])

    pltpu.sync_copy(ii_hbm.at[wid], ii_v)
    pltpu.sync_copy(jj_hbm.at[wid], jj_v)
    pltpu.sync_copy(attr_hbm.at[wid], attr_v)

    @pl.loop(0, RC)
    def _(r):
        for d in range(CH // 16):
            sl = pl.ds(d * 16, 16)
            iv = ii_v[r, sl]
            jv = jj_v[r, sl]
            mask = iv != jv
            ones_v[r, sl] = jnp.where(mask, 1.0, 0.0).astype(jnp.float32)
            attrm_v[r, sl] = jnp.where(mask, attr_v[r, sl], 0.0)

    plsc.subcore_barrier()

    @pl.loop(0, RC)
    def _(r):
        pltpu.sync_copy(ones_v.at[r], deg_acc.at[ii_v.at[r]], add=True)
        pltpu.sync_copy(ones_v.at[r], deg_acc.at[jj_v.at[r]], add=True)

    plsc.subcore_barrier()

    base = sid * ROWS_PER_TILE
    pltpu.sync_copy(deg_acc.at[pl.ds(base, ROWS_PER_TILE)],
                    degp_hbm.at[cid].at[pl.ds(base, ROWS_PER_TILE)])
    pltpu.sync_copy(attrm_v, attrm_hbm.at[wid])


@functools.partial(
    pl.kernel,
    out_type=jax.ShapeDtypeStruct((2, NP, D), jnp.float32),
    mesh=_mesh,
    scratch_types=[
        pltpu.VMEM((RC // 5, CH), jnp.int32),
        pltpu.VMEM((RC // 5, CH), jnp.int32),
        pltpu.VMEM((RC // 5, CH), jnp.float32),
        pltpu.VMEM((CH, D), jnp.float32),
        pltpu.VMEM((CH, D), jnp.float32),
        pltpu.VMEM((16, D), jnp.float32),
        pltpu.VMEM_SHARED((NP, D), jnp.float32),
        pltpu.SemaphoreType.DMA,
        pltpu.SemaphoreType.DMA,
        pltpu.SemaphoreType.DMA,
        pltpu.SemaphoreType.DMA,
    ],
    compiler_params=_sc_params,
)
def _edge_pass_kernel(ii_hbm, jj_hbm, attrm_hbm, g_hbm, out_hbm,
                      ii_v, jj_v, attrm_v, rows_a, rows_b, zbuf, acc,
                      sem_a, sem_b, sem_sa, sem_sb):
    cid = lax.axis_index("c")
    sid = lax.axis_index("s")
    wid = cid * 16 + sid
    cpb = RC // 5

    @pl.loop(0, 16)
    def _(r):
        for d in range(8):
            zbuf[r, pl.ds(d * 16, 16)] = jnp.zeros((16,), jnp.float32)

    @pl.loop(0, ROWS_PER_TILE // 16)
    def _(k):
        pltpu.sync_copy(zbuf, acc.at[pl.ds(sid * ROWS_PER_TILE + k * 16, 16)])

    plsc.subcore_barrier()

    def _start_gather(c, buf, sem):
        pltpu.async_copy(g_hbm.at[jj_v.at[c]], buf, sem)

    def _wait_gather(buf, sem):
        pltpu.make_async_copy(g_hbm.at[pl.ds(0, CH)], buf, sem).wait()

    def _scale(c, buf):
        @pl.loop(0, CH, step=2)
        def _(r):
            for u in range(2):
                val = plsc.load_gather(
                    attrm_v,
                    [jnp.full((16,), c, jnp.int32),
                     jnp.full((16,), r + u, jnp.int32)],
                )
                for d in range(8):
                    sl = pl.ds(d * 16, 16)
                    buf[r + u, sl] = buf[r + u, sl] * val

    def _start_scatter(c, buf, sem):
        pltpu.async_copy(buf, acc.at[ii_v.at[c]], sem, add=True)

    def _wait_scatter(buf, sem):
        pltpu.make_async_copy(buf, acc.at[pl.ds(0, CH)], sem).wait()

    @pl.loop(0, 5)
    def _(ob):
        pltpu.sync_copy(ii_hbm.at[wid].at[pl.ds(ob * cpb, cpb)], ii_v)
        pltpu.sync_copy(jj_hbm.at[wid].at[pl.ds(ob * cpb, cpb)], jj_v)
        pltpu.sync_copy(attrm_hbm.at[wid].at[pl.ds(ob * cpb, cpb)], attrm_v)

        _start_gather(0, rows_a, sem_a)
        _start_gather(1, rows_b, sem_b)

        @pl.loop(0, cpb, step=2)
        def _(c):
            _wait_gather(rows_a, sem_a)
            _scale(c, rows_a)
            _start_scatter(c, rows_a, sem_sa)

            _wait_gather(rows_b, sem_b)
            _scale(c + 1, rows_b)
            _start_scatter(c + 1, rows_b, sem_sb)

            _wait_scatter(rows_a, sem_sa)

            @pl.when(c + 2 < cpb)
            def _():
                _start_gather(c + 2, rows_a, sem_a)

            _wait_scatter(rows_b, sem_sb)

            @pl.when(c + 3 < cpb)
            def _():
                _start_gather(c + 3, rows_b, sem_b)

    plsc.subcore_barrier()

    base = sid * ROWS_PER_TILE
    pltpu.sync_copy(acc.at[pl.ds(base, ROWS_PER_TILE)],
                    out_hbm.at[cid].at[pl.ds(base, ROWS_PER_TILE)])


def _linear_body(x_ref, wt_ref, b_ref, h_ref):
    h_ref[...] = (
        jnp.dot(x_ref[...], wt_ref[...], preferred_element_type=jnp.float32)
        + b_ref[...]
    )


def _norm_body(p0_ref, p1_ref, h_ref, g_ref, selfd_ref, isdf_ref):
    deg = p0_ref[...] + p1_ref[...] + 2.0
    isd = lax.rsqrt(deg)
    h = h_ref[...]
    g_ref[...] = isd * h
    selfd_ref[...] = h / deg
    isdf_ref[...] = jnp.broadcast_to(isd, h.shape)


def _combine_body(q0_ref, q1_ref, isdf_ref, selfd_ref, o_ref):
    o_ref[...] = jnp.maximum(
        isdf_ref[...] * (q0_ref[...] + q1_ref[...]) + selfd_ref[...], 0.0
    )


def kernel(vertices, edges, edge_attr, W, b):
    ii = edges[0].astype(jnp.int32)
    jj = edges[1].astype(jnp.int32)
    pad_e = NW * EPW - E
    pad_idx = jnp.full((pad_e,), N, jnp.int32)
    ii_p = jnp.concatenate([ii, pad_idx]).reshape(NW, RC, CH)
    jj_p = jnp.concatenate([jj, pad_idx]).reshape(NW, RC, CH)
    attr_p = jnp.concatenate(
        [edge_attr.astype(jnp.float32), jnp.zeros((pad_e,), jnp.float32)]
    ).reshape(NW, RC, CH)

    v_pad = jnp.concatenate(
        [vertices, jnp.zeros((NP - N, D), jnp.float32)], axis=0
    )
    w_t = W.T
    b2d = b.reshape(1, D)

    degp, attrm = _degree_kernel(ii_p, jj_p, attr_p)

    blk = 512
    grid = (NP // blk,)
    h_pad = pl.pallas_call(
        _linear_body,
        grid=grid,
        in_specs=[
            pl.BlockSpec((blk, D), lambda i: (i, 0)),
            pl.BlockSpec((D, D), lambda i: (0, 0)),
            pl.BlockSpec((1, D), lambda i: (0, 0)),
        ],
        out_specs=pl.BlockSpec((blk, D), lambda i: (i, 0)),
        out_shape=jax.ShapeDtypeStruct((NP, D), jnp.float32),
    )(v_pad, w_t, b2d)

    p0 = degp[0].reshape(NP, 1)
    p1 = degp[1].reshape(NP, 1)
    col_spec = pl.BlockSpec((blk, 1), lambda i: (i, 0))
    mat_spec = pl.BlockSpec((blk, D), lambda i: (i, 0))
    g, selfd, isdf = pl.pallas_call(
        _norm_body,
        grid=grid,
        in_specs=[col_spec, col_spec, mat_spec],
        out_specs=[mat_spec, mat_spec, mat_spec],
        out_shape=[
            jax.ShapeDtypeStruct((NP, D), jnp.float32),
            jax.ShapeDtypeStruct((NP, D), jnp.float32),
            jax.ShapeDtypeStruct((NP, D), jnp.float32),
        ],
    )(p0, p1, h_pad)

    partials = _edge_pass_kernel(ii_p, jj_p, attrm, g)

    out_pad = pl.pallas_call(
        _combine_body,
        grid=grid,
        in_specs=[mat_spec, mat_spec, mat_spec, mat_spec],
        out_specs=mat_spec,
        out_shape=jax.ShapeDtypeStruct((NP, D), jnp.float32),
    )(partials[0], partials[1], isdf, selfd)

    return out_pad[:N]


# speedup vs baseline: 1.0898x; 1.0898x over previous
import dataclasses
import functools

import jax
import jax.numpy as jnp
from jax import lax
from jax.experimental import pallas as pl
from jax.experimental.pallas import tpu as pltpu
from jax.experimental.pallas import tpu_sc as plsc

N = 10000
E = 320000
D = 128
NP = 10240
NW = 32
EPW = 10240
CH = 64
RC = EPW // CH
ROWS_PER_TILE = NP // 16

_mesh = plsc.VectorSubcoreMesh(core_axis_name="c", subcore_axis_name="s")

_sc_params = pltpu.CompilerParams()
if "needs_layout_passes" in pltpu.CompilerParams.__dataclass_fields__:
    _sc_params = dataclasses.replace(_sc_params, needs_layout_passes=False)


@functools.partial(
    pl.kernel,
    out_type=[
        jax.ShapeDtypeStruct((2, NP), jnp.float32),
        jax.ShapeDtypeStruct((NW, RC, CH), jnp.float32),
    ],
    mesh=_mesh,
    scratch_types=[
        pltpu.VMEM((RC, CH), jnp.int32),
        pltpu.VMEM((RC, CH), jnp.int32),
        pltpu.VMEM((RC, CH), jnp.float32),
        pltpu.VMEM((RC, CH), jnp.float32),
        pltpu.VMEM((RC, CH), jnp.float32),
        pltpu.VMEM((ROWS_PER_TILE,), jnp.float32),
        pltpu.VMEM_SHARED((NP,), jnp.float32),
    ],
)
def _degree_kernel(ii_hbm, jj_hbm, attr_hbm, degp_hbm, attrm_hbm,
                   ii_v, jj_v, attr_v, ones_v, attrm_v, zbuf, deg_acc):
    cid = lax.axis_index("c")
    sid = lax.axis_index("s")
    wid = cid * 16 + sid

    @pl.loop(0, ROWS_PER_TILE, step=16)
    def _(k):
        zbuf[pl.ds(k, 16)] = jnp.zeros((16,), jnp.float32)

    pltpu.sync_copy(zbuf, deg_acc.at[pl.ds(sid * ROWS_PER_TILE, ROWS_PER_TILE)])

    pltpu.sync_copy(ii_hbm.at[wid], ii_v)
    pltpu.sync_copy(jj_hbm.at[wid], jj_v)
    pltpu.sync_copy(attr_hbm.at[wid], attr_v)

    @pl.loop(0, RC)
    def _(r):
        for d in range(CH // 16):
            sl = pl.ds(d * 16, 16)
            iv = ii_v[r, sl]
            jv = jj_v[r, sl]
            mask = iv != jv
            ones_v[r, sl] = jnp.where(mask, 1.0, 0.0).astype(jnp.float32)
            attrm_v[r, sl] = jnp.where(mask, attr_v[r, sl], 0.0)

    plsc.subcore_barrier()

    @pl.loop(0, RC)
    def _(r):
        pltpu.sync_copy(ones_v.at[r], deg_acc.at[ii_v.at[r]], add=True)
        pltpu.sync_copy(ones_v.at[r], deg_acc.at[jj_v.at[r]], add=True)

    plsc.subcore_barrier()

    base = sid * ROWS_PER_TILE
    pltpu.sync_copy(deg_acc.at[pl.ds(base, ROWS_PER_TILE)],
                    degp_hbm.at[cid].at[pl.ds(base, ROWS_PER_TILE)])
    pltpu.sync_copy(attrm_v, attrm_hbm.at[wid])


@functools.partial(
    pl.kernel,
    out_type=jax.ShapeDtypeStruct((2, NP, D), jnp.float32),
    mesh=_mesh,
    scratch_types=[
        pltpu.VMEM((RC // 5, CH), jnp.int32),
        pltpu.VMEM((RC // 5, CH), jnp.int32),
        pltpu.VMEM((RC // 5, CH), jnp.float32),
        pltpu.VMEM((CH, D), jnp.float32),
        pltpu.VMEM((CH, D), jnp.float32),
        pltpu.VMEM((16, D), jnp.float32),
        pltpu.VMEM_SHARED((NP, D), jnp.float32),
        pltpu.SemaphoreType.DMA,
        pltpu.SemaphoreType.DMA,
        pltpu.SemaphoreType.DMA,
        pltpu.SemaphoreType.DMA,
    ],
    compiler_params=_sc_params,
)
def _edge_pass_kernel(ii_hbm, jj_hbm, attrm_hbm, g_hbm, out_hbm,
                      ii_v, jj_v, attrm_v, rows_a, rows_b, zbuf, acc,
                      sem_a, sem_b, sem_sa, sem_sb):
    cid = lax.axis_index("c")
    sid = lax.axis_index("s")
    wid = cid * 16 + sid
    cpb = RC // 5

    @pl.loop(0, 16)
    def _(r):
        for d in range(8):
            zbuf[r, pl.ds(d * 16, 16)] = jnp.zeros((16,), jnp.float32)

    @pl.loop(0, ROWS_PER_TILE // 16)
    def _(k):
        pltpu.sync_copy(zbuf, acc.at[pl.ds(sid * ROWS_PER_TILE + k * 16, 16)])

    plsc.subcore_barrier()

    def _start_gather(c, buf, sem):
        pltpu.async_copy(g_hbm.at[jj_v.at[c]], buf, sem)

    def _wait_gather(buf, sem):
        pltpu.make_async_copy(g_hbm.at[pl.ds(0, CH)], buf, sem).wait()

    def _scale(c, buf):
        @pl.loop(0, CH, step=4)
        def _(r):
            for u in range(4):
                val = plsc.load_gather(
                    attrm_v,
                    [jnp.full((16,), c, jnp.int32),
                     jnp.full((16,), r + u, jnp.int32)],
                )
                for d in range(8):
                    sl = pl.ds(d * 16, 16)
                    buf[r + u, sl] = buf[r + u, sl] * val

    def _start_scatter(c, buf, sem):
        pltpu.async_copy(buf, acc.at[ii_v.at[c]], sem, add=True)

    def _wait_scatter(buf, sem):
        pltpu.make_async_copy(buf, acc.at[pl.ds(0, CH)], sem).wait()

    @pl.loop(0, 5)
    def _(ob):
        pltpu.sync_copy(ii_hbm.at[wid].at[pl.ds(ob * cpb, cpb)], ii_v)
        pltpu.sync_copy(jj_hbm.at[wid].at[pl.ds(ob * cpb, cpb)], jj_v)
        pltpu.sync_copy(attrm_hbm.at[wid].at[pl.ds(ob * cpb, cpb)], attrm_v)

        _start_gather(0, rows_a, sem_a)
        _start_gather(1, rows_b, sem_b)

        @pl.loop(0, cpb, step=2)
        def _(c):
            _wait_gather(rows_a, sem_a)
            _scale(c, rows_a)
            pltpu.sync_copy(rows_a, acc.at[ii_v.at[c]], add=True)

            @pl.when(c + 2 < cpb)
            def _():
                _start_gather(c + 2, rows_a, sem_a)

            _wait_gather(rows_b, sem_b)
            _scale(c + 1, rows_b)
            pltpu.sync_copy(rows_b, acc.at[ii_v.at[c + 1]], add=True)

            @pl.when(c + 3 < cpb)
            def _():
                _start_gather(c + 3, rows_b, sem_b)

    plsc.subcore_barrier()

    base = sid * ROWS_PER_TILE
    pltpu.sync_copy(acc.at[pl.ds(base, ROWS_PER_TILE)],
                    out_hbm.at[cid].at[pl.ds(base, ROWS_PER_TILE)])


def _linear_body(x_ref, wt_ref, b_ref, h_ref):
    h_ref[...] = (
        jnp.dot(x_ref[...], wt_ref[...], preferred_element_type=jnp.float32)
        + b_ref[...]
    )


def _norm_body(p0_ref, p1_ref, h_ref, g_ref, selfd_ref, isdf_ref):
    deg = p0_ref[...] + p1_ref[...] + 2.0
    isd = lax.rsqrt(deg)
    h = h_ref[...]
    g_ref[...] = isd * h
    selfd_ref[...] = h / deg
    isdf_ref[...] = jnp.broadcast_to(isd, h.shape)


def _combine_body(q0_ref, q1_ref, isdf_ref, selfd_ref, o_ref):
    o_ref[...] = jnp.maximum(
        isdf_ref[...] * (q0_ref[...] + q1_ref[...]) + selfd_ref[...], 0.0
    )


def kernel(vertices, edges, edge_attr, W, b):
    ii = edges[0].astype(jnp.int32)
    jj = edges[1].astype(jnp.int32)
    pad_e = NW * EPW - E
    pad_idx = jnp.full((pad_e,), N, jnp.int32)
    ii_p = jnp.concatenate([ii, pad_idx]).reshape(NW, RC, CH)
    jj_p = jnp.concatenate([jj, pad_idx]).reshape(NW, RC, CH)
    attr_p = jnp.concatenate(
        [edge_attr.astype(jnp.float32), jnp.zeros((pad_e,), jnp.float32)]
    ).reshape(NW, RC, CH)

    v_pad = jnp.concatenate(
        [vertices, jnp.zeros((NP - N, D), jnp.float32)], axis=0
    )
    w_t = W.T
    b2d = b.reshape(1, D)

    degp, attrm = _degree_kernel(ii_p, jj_p, attr_p)

    blk = 512
    grid = (NP // blk,)
    h_pad = pl.pallas_call(
        _linear_body,
        grid=grid,
        in_specs=[
            pl.BlockSpec((blk, D), lambda i: (i, 0)),
            pl.BlockSpec((D, D), lambda i: (0, 0)),
            pl.BlockSpec((1, D), lambda i: (0, 0)),
        ],
        out_specs=pl.BlockSpec((blk, D), lambda i: (i, 0)),
        out_shape=jax.ShapeDtypeStruct((NP, D), jnp.float32),
    )(v_pad, w_t, b2d)

    p0 = degp[0].reshape(NP, 1)
    p1 = degp[1].reshape(NP, 1)
    col_spec = pl.BlockSpec((blk, 1), lambda i: (i, 0))
    mat_spec = pl.BlockSpec((blk, D), lambda i: (i, 0))
    g, selfd, isdf = pl.pallas_call(
        _norm_body,
        grid=grid,
        in_specs=[col_spec, col_spec, mat_spec],
        out_specs=[mat_spec, mat_spec, mat_spec],
        out_shape=[
            jax.ShapeDtypeStruct((NP, D), jnp.float32),
            jax.ShapeDtypeStruct((NP, D), jnp.float32),
            jax.ShapeDtypeStruct((NP, D), jnp.float32),
        ],
    )(p0, p1, h_pad)

    partials = _edge_pass_kernel(ii_p, jj_p, attrm, g)

    out_pad = pl.pallas_call(
        _combine_body,
        grid=grid,
        in_specs=[mat_spec, mat_spec, mat_spec, mat_spec],
        out_specs=mat_spec,
        out_shape=jax.ShapeDtypeStruct((NP, D), jnp.float32),
    )(partials[0], partials[1], isdf, selfd)

    return out_pad[:N]


# speedup vs baseline: 1.8269x; 1.6764x over previous
import dataclasses
import functools

import jax
import jax.numpy as jnp
from jax import lax
from jax.experimental import pallas as pl
from jax.experimental.pallas import tpu as pltpu
from jax.experimental.pallas import tpu_sc as plsc

N = 10000
E = 320000
D = 128
NP = 10240
NW = 32
EPW = 10240
CH = 64
RC = EPW // CH
ROWS_PER_TILE = NP // 16

_mesh = plsc.VectorSubcoreMesh(core_axis_name="c", subcore_axis_name="s")

_sc_params = pltpu.CompilerParams()
if "needs_layout_passes" in pltpu.CompilerParams.__dataclass_fields__:
    _sc_params = dataclasses.replace(_sc_params, needs_layout_passes=False)


@functools.partial(
    pl.kernel,
    out_type=[
        jax.ShapeDtypeStruct((2, NP), jnp.float32),
        jax.ShapeDtypeStruct((NW, RC, CH), jnp.float32),
    ],
    mesh=_mesh,
    scratch_types=[
        pltpu.VMEM((RC, CH), jnp.int32),
        pltpu.VMEM((RC, CH), jnp.int32),
        pltpu.VMEM((RC, CH), jnp.float32),
        pltpu.VMEM((RC, CH), jnp.float32),
        pltpu.VMEM((RC, CH), jnp.float32),
        pltpu.VMEM((ROWS_PER_TILE,), jnp.float32),
        pltpu.VMEM_SHARED((NP,), jnp.float32),
    ],
)
def _degree_kernel(ii_hbm, jj_hbm, attr_hbm, degp_hbm, attrm_hbm,
                   ii_v, jj_v, attr_v, ones_v, attrm_v, zbuf, deg_acc):
    cid = lax.axis_index("c")
    sid = lax.axis_index("s")
    wid = cid * 16 + sid

    @pl.loop(0, ROWS_PER_TILE, step=16)
    def _(k):
        zbuf[pl.ds(k, 16)] = jnp.zeros((16,), jnp.float32)

    pltpu.sync_copy(zbuf, deg_acc.at[pl.ds(sid * ROWS_PER_TILE, ROWS_PER_TILE)])

    pltpu.sync_copy(ii_hbm.at[wid], ii_v)
    pltpu.sync_copy(jj_hbm.at[wid], jj_v)
    pltpu.sync_copy(attr_hbm.at[wid], attr_v)

    @pl.loop(0, RC)
    def _(r):
        for d in range(CH // 16):
            sl = pl.ds(d * 16, 16)
            iv = ii_v[r, sl]
            jv = jj_v[r, sl]
            mask = iv != jv
            ones_v[r, sl] = jnp.where(mask, 1.0, 0.0).astype(jnp.float32)
            attrm_v[r, sl] = jnp.where(mask, attr_v[r, sl], 0.0)

    plsc.subcore_barrier()

    @pl.loop(0, RC)
    def _(r):
        pltpu.sync_copy(ones_v.at[r], deg_acc.at[ii_v.at[r]], add=True)
        pltpu.sync_copy(ones_v.at[r], deg_acc.at[jj_v.at[r]], add=True)

    plsc.subcore_barrier()

    base = sid * ROWS_PER_TILE
    pltpu.sync_copy(deg_acc.at[pl.ds(base, ROWS_PER_TILE)],
                    degp_hbm.at[cid].at[pl.ds(base, ROWS_PER_TILE)])
    pltpu.sync_copy(attrm_v, attrm_hbm.at[wid])


@functools.partial(
    pl.kernel,
    out_type=jax.ShapeDtypeStruct((2, NP, D), jnp.float32),
    mesh=_mesh,
    scratch_types=[
        pltpu.VMEM((RC // 5, CH), jnp.int32),
        pltpu.VMEM((RC // 5, CH), jnp.int32),
        pltpu.VMEM((RC // 5, CH), jnp.float32),
        pltpu.VMEM((CH, D), jnp.float32),
        pltpu.VMEM((CH, D), jnp.float32),
        pltpu.VMEM((16, D), jnp.float32),
        pltpu.VMEM_SHARED((NP, D), jnp.float32),
        pltpu.SemaphoreType.DMA,
        pltpu.SemaphoreType.DMA,
        pltpu.SemaphoreType.DMA,
        pltpu.SemaphoreType.DMA,
    ],
    compiler_params=_sc_params,
)
def _edge_pass_kernel(ii_hbm, jj_hbm, attrm_hbm, g_hbm, out_hbm,
                      ii_v, jj_v, attrm_v, rows_a, rows_b, zbuf, acc,
                      sem_a, sem_b, sem_sa, sem_sb):
    cid = lax.axis_index("c")
    sid = lax.axis_index("s")
    wid = cid * 16 + sid
    cpb = RC // 5

    @pl.loop(0, 16)
    def _(r):
        for d in range(8):
            zbuf[r, pl.ds(d * 16, 16)] = jnp.zeros((16,), jnp.float32)

    @pl.loop(0, ROWS_PER_TILE // 16)
    def _(k):
        pltpu.sync_copy(zbuf, acc.at[pl.ds(sid * ROWS_PER_TILE + k * 16, 16)])

    plsc.subcore_barrier()

    def _start_gather(c, buf, sem):
        pltpu.async_copy(g_hbm.at[jj_v.at[c]], buf, sem)

    def _wait_gather(buf, sem):
        pltpu.make_async_copy(g_hbm.at[pl.ds(0, CH)], buf, sem).wait()

    def _scale(c, buf):
        @pl.loop(0, CH, step=4)
        def _(r):
            for u in range(4):
                val = plsc.load_gather(
                    attrm_v,
                    [jnp.full((16,), c, jnp.int32),
                     jnp.full((16,), r + u, jnp.int32)],
                )
                for d in range(8):
                    sl = pl.ds(d * 16, 16)
                    buf[r + u, sl] = buf[r + u, sl] * val

    def _start_scatter(c, buf, sem):
        pltpu.async_copy(buf, acc.at[ii_v.at[c]], sem, add=True)

    def _wait_scatter(buf, sem):
        pltpu.make_async_copy(buf, acc.at[pl.ds(0, CH)], sem).wait()

    @pl.loop(0, 5)
    def _(ob):
        pltpu.sync_copy(ii_hbm.at[wid].at[pl.ds(ob * cpb, cpb)], ii_v)
        pltpu.sync_copy(jj_hbm.at[wid].at[pl.ds(ob * cpb, cpb)], jj_v)
        pltpu.sync_copy(attrm_hbm.at[wid].at[pl.ds(ob * cpb, cpb)], attrm_v)

        _start_gather(0, rows_a, sem_a)
        _start_gather(1, rows_b, sem_b)

        @pl.loop(0, cpb, step=2)
        def _(c):
            _wait_gather(rows_a, sem_a)
            _scale(c, rows_a)
            pltpu.sync_copy(rows_a, acc.at[ii_v.at[c]], add=True)

            @pl.when(c + 2 < cpb)
            def _():
                _start_gather(c + 2, rows_a, sem_a)

            _wait_gather(rows_b, sem_b)
            _scale(c + 1, rows_b)
            pltpu.sync_copy(rows_b, acc.at[ii_v.at[c + 1]], add=True)

            @pl.when(c + 3 < cpb)
            def _():
                _start_gather(c + 3, rows_b, sem_b)

    plsc.subcore_barrier()

    base = sid * ROWS_PER_TILE
    pltpu.sync_copy(acc.at[pl.ds(base, ROWS_PER_TILE)],
                    out_hbm.at[cid].at[pl.ds(base, ROWS_PER_TILE)])


def _linear_body(x_ref, wt_ref, b_ref, h_ref):
    h_ref[...] = (
        jnp.dot(x_ref[...], wt_ref[...], preferred_element_type=jnp.float32)
        + b_ref[...]
    )


def _norm_body(p0_ref, p1_ref, h_ref, g_ref, selfd_ref, isdf_ref):
    deg = p0_ref[...] + p1_ref[...] + 2.0
    isd = lax.rsqrt(deg)
    h = h_ref[...]
    g_ref[...] = isd * h
    selfd_ref[...] = h / deg
    isdf_ref[...] = jnp.broadcast_to(isd, h.shape)


def _combine_body(q0_ref, q1_ref, isdf_ref, selfd_ref, o_ref):
    o_ref[...] = jnp.maximum(
        isdf_ref[...] * (q0_ref[...] + q1_ref[...]) + selfd_ref[...], 0.0
    )


def kernel(vertices, edges, edge_attr, W, b):
    ii = edges[0].astype(jnp.int32)
    jj = edges[1].astype(jnp.int32)
    ppw = EPW - E // NW
    pad_idx = jnp.broadcast_to(
        N + jnp.arange(ppw, dtype=jnp.int32)[None, :], (NW, ppw)
    )

    def _shard(x, pad):
        return jnp.concatenate(
            [x.reshape(NW, E // NW), pad], axis=1
        ).reshape(NW, RC, CH)

    ii_p = _shard(ii, pad_idx)
    jj_p = _shard(jj, pad_idx)
    attr_p = _shard(
        edge_attr.astype(jnp.float32), jnp.zeros((NW, ppw), jnp.float32)
    )

    v_pad = jnp.concatenate(
        [vertices, jnp.zeros((NP - N, D), jnp.float32)], axis=0
    )
    w_t = W.T
    b2d = b.reshape(1, D)

    degp, attrm = _degree_kernel(ii_p, jj_p, attr_p)

    blk = 512
    grid = (NP // blk,)
    h_pad = pl.pallas_call(
        _linear_body,
        grid=grid,
        in_specs=[
            pl.BlockSpec((blk, D), lambda i: (i, 0)),
            pl.BlockSpec((D, D), lambda i: (0, 0)),
            pl.BlockSpec((1, D), lambda i: (0, 0)),
        ],
        out_specs=pl.BlockSpec((blk, D), lambda i: (i, 0)),
        out_shape=jax.ShapeDtypeStruct((NP, D), jnp.float32),
    )(v_pad, w_t, b2d)

    p0 = degp[0].reshape(NP, 1)
    p1 = degp[1].reshape(NP, 1)
    col_spec = pl.BlockSpec((blk, 1), lambda i: (i, 0))
    mat_spec = pl.BlockSpec((blk, D), lambda i: (i, 0))
    g, selfd, isdf = pl.pallas_call(
        _norm_body,
        grid=grid,
        in_specs=[col_spec, col_spec, mat_spec],
        out_specs=[mat_spec, mat_spec, mat_spec],
        out_shape=[
            jax.ShapeDtypeStruct((NP, D), jnp.float32),
            jax.ShapeDtypeStruct((NP, D), jnp.float32),
            jax.ShapeDtypeStruct((NP, D), jnp.float32),
        ],
    )(p0, p1, h_pad)

    partials = _edge_pass_kernel(ii_p, jj_p, attrm, g)

    out_pad = pl.pallas_call(
        _combine_body,
        grid=grid,
        in_specs=[mat_spec, mat_spec, mat_spec, mat_spec],
        out_specs=mat_spec,
        out_shape=jax.ShapeDtypeStruct((NP, D), jnp.float32),
    )(partials[0], partials[1], isdf, selfd)

    return out_pad[:N]
